# Initial kernel scaffold; baseline (speedup 1.0000x reference)
import jax
import jax.numpy as jnp
from jax.experimental import pallas as pl


def kernel(x, W1, g1, b1, W2, g2, b2, W5, g5, b5):
    raise NotImplementedError("write your pallas kernel here")


# speedup vs baseline: 7.3174x; 7.3174x over previous
import functools

import jax
import jax.numpy as jnp
from jax import lax
from jax.experimental import pallas as pl
from jax.experimental.pallas import tpu as pltpu
from jax.experimental.pallas import tpu_sc as plsc

KNN = 20
EPS = 1e-5
B = 8
N = 2048
BN = B * N
F = 64
EMB = 1024

NC, NS = 2, 16
NW = NC * NS
PTS_PER_W = BN // NW
CHUNK = 64
NCHUNK = PTS_PER_W // CHUNK

R_KNN = 256
R_FIN = 256

NEG_BIG = -3.0e38


def _lrelu(y):
    return jnp.where(y >= 0, y, 0.2 * y)


def _knn_body(xf_ref, xr_ref, idx_ref):
    b = pl.program_id(0)
    xf = xf_ref[0]
    xr = xr_ref[0]
    xxc = jnp.sum(xf * xf, axis=1, keepdims=True).reshape(1, xf.shape[0])
    xxr = jnp.sum(xr * xr, axis=1, keepdims=True)
    g = lax.dot_general(xr, xf, (((1,), (1,)), ((), ())),
                        preferred_element_type=jnp.float32)
    d = 2.0 * g - xxr - xxc
    iota = lax.broadcasted_iota(jnp.int32, d.shape, 1)
    cols = []
    for _ in range(KNN):
        m = jnp.max(d, axis=1, keepdims=True)
        sel = jnp.where(d >= m, iota, N)
        idx_t = jnp.min(sel, axis=1, keepdims=True)
        cols.append(idx_t)
        d = jnp.where(iota == idx_t, NEG_BIG, d)
    idx_ref[0] = jnp.concatenate(cols, axis=1) + b * N


def _knn(xt, c):
    return pl.pallas_call(
        _knn_body,
        grid=(B, N // R_KNN),
        in_specs=[
            pl.BlockSpec((1, N, c), lambda b, t: (b, 0, 0)),
            pl.BlockSpec((1, R_KNN, c), lambda b, t: (b, t, 0)),
        ],
        out_specs=pl.BlockSpec((1, R_KNN, KNN), lambda b, t: (b, t, 0)),
        out_shape=jax.ShapeDtypeStruct((B, N, KNN), jnp.int32),
    )(xt, xt)


def _sc1_body(tab_hbm, idx_hbm, g_hbm, idx_v, rows_v, sem):
    wid = lax.axis_index("s") * NC + lax.axis_index("c")

    def chunk_body(ch, carry):
        p0 = wid * PTS_PER_W + ch * CHUNK
        pltpu.sync_copy(idx_hbm.at[pl.ds(p0 * KNN, CHUNK * KNN)], idx_v)
        pltpu.async_copy(tab_hbm.at[idx_v], rows_v, sem).wait()
        pltpu.sync_copy(rows_v, g_hbm.at[pl.ds(p0 * KNN, CHUNK * KNN)])
        return carry

    lax.fori_loop(0, NCHUNK, chunk_body, 0)


@functools.lru_cache(maxsize=None)
def _sc_gather_rows_fn():
    return pl.kernel(
        _sc1_body,
        out_type=jax.ShapeDtypeStruct((BN * KNN, 16), jnp.float32),
        mesh=plsc.VectorSubcoreMesh(core_axis_name="c", subcore_axis_name="s",
                                    num_cores=NC, num_subcores=NS),
        scratch_types=[
            pltpu.VMEM((CHUNK * KNN,), jnp.int32),
            pltpu.VMEM((CHUNK * KNN, 16), jnp.float32),
            pltpu.SemaphoreType.DMA,
        ],
        compiler_params=pltpu.CompilerParams(use_tc_tiling_on_sc=False),
    )


def _sc_gather_rows(tab, idx):
    return _sc_gather_rows_fn()(tab, idx)


def _conv1_body(g_ref, xt_ref, w_ref, m_ref, ysum_ref, ysq_ref):
    ti = pl.program_id(1)
    first = jnp.logical_and(pl.program_id(0) == 0, ti == 0)
    r = R_KNN
    xj = g_ref[0]
    xi = xt_ref[0]
    xib = jnp.broadcast_to(xi[:, None, :], (r, KNN, 16)).reshape(r * KNN, 16)
    f = jnp.concatenate(
        [xj[:, 0:3] - xib[:, 0:3], xib[:, 0:3],
         jnp.zeros((r * KNN, 2), jnp.float32)], axis=1)
    y = jnp.dot(f.astype(jnp.bfloat16), w_ref[...],
                preferred_element_type=jnp.float32)
    m_ref[0] = jnp.max(y.reshape(r, KNN, F), axis=1)
    ts = jnp.sum(y, axis=0, keepdims=True)
    tq = jnp.sum(y * y, axis=0, keepdims=True)

    @pl.when(first)
    def _():
        ysum_ref[...] = ts
        ysq_ref[...] = tq

    @pl.when(jnp.logical_not(first))
    def _():
        ysum_ref[...] = ysum_ref[...] + ts
        ysq_ref[...] = ysq_ref[...] + tq


def _conv1(g1, xtp, w1t8):
    return pl.pallas_call(
        _conv1_body,
        grid=(B, N // R_KNN),
        in_specs=[
            pl.BlockSpec((1, R_KNN * KNN, 16), lambda b, t: (b, t, 0)),
            pl.BlockSpec((1, R_KNN, 16), lambda b, t: (b, t, 0)),
            pl.BlockSpec((8, F), lambda b, t: (0, 0)),
        ],
        out_specs=(
            pl.BlockSpec((1, R_KNN, F), lambda b, t: (b, t, 0)),
            pl.BlockSpec((1, F), lambda b, t: (0, 0)),
            pl.BlockSpec((1, F), lambda b, t: (0, 0)),
        ),
        out_shape=(jax.ShapeDtypeStruct((B, N, F), jnp.float32),
                   jax.ShapeDtypeStruct((1, F), jnp.float32),
                   jax.ShapeDtypeStruct((1, F), jnp.float32)),
    )(g1, xtp, w1t8)


def _sc_body(tab_hbm, idx_hbm, m_hbm, s_hbm, q_hbm, idx_v, rows_v, ms_v, q_v,
             sem):
    wid = lax.axis_index("s") * NC + lax.axis_index("c")
    zero = jnp.zeros((16,), jnp.float32)

    def chunk_body(ch, qcarry):
        p0 = wid * PTS_PER_W + ch * CHUNK
        pltpu.sync_copy(idx_hbm.at[pl.ds(p0 * KNN, CHUNK * KNN)], idx_v)
        pltpu.async_copy(tab_hbm.at[idx_v], rows_v, sem).wait()

        def pt_body(p, qc):
            q0, q1, q2, q3 = qc
            row0 = p * KNN
            m = [jnp.full((16,), NEG_BIG, jnp.float32) for _ in range(4)]
            s = [zero for _ in range(4)]
            for r in range(KNN):
                for c in range(4):
                    v = rows_v[row0 + r, pl.ds(c * 16, 16)]
                    m[c] = jnp.maximum(m[c], v)
                    s[c] = s[c] + v
                    if c == 0:
                        q0 = q0 + v * v
                    elif c == 1:
                        q1 = q1 + v * v
                    elif c == 2:
                        q2 = q2 + v * v
                    else:
                        q3 = q3 + v * v
            for c in range(4):
                ms_v[p, pl.ds(c * 16, 16)] = m[c]
                ms_v[p + CHUNK, pl.ds(c * 16, 16)] = s[c]
            return (q0, q1, q2, q3)

        qcarry = lax.fori_loop(0, CHUNK, pt_body, qcarry)
        pltpu.sync_copy(ms_v.at[pl.ds(0, CHUNK)], m_hbm.at[pl.ds(p0, CHUNK)])
        pltpu.sync_copy(ms_v.at[pl.ds(CHUNK, CHUNK)],
                        s_hbm.at[pl.ds(p0, CHUNK)])
        return qcarry

    q = lax.fori_loop(0, NCHUNK, chunk_body, (zero, zero, zero, zero))
    for c in range(4):
        q_v[0, pl.ds(c * 16, 16)] = q[c]
    pltpu.sync_copy(q_v, q_hbm.at[pl.ds(wid, 1)])


@functools.lru_cache(maxsize=None)
def _sc_gather_fn():
    return pl.kernel(
        _sc_body,
        out_type=(
            jax.ShapeDtypeStruct((BN, F), jnp.float32),
            jax.ShapeDtypeStruct((BN, F), jnp.float32),
            jax.ShapeDtypeStruct((NW, F), jnp.float32),
        ),
        mesh=plsc.VectorSubcoreMesh(core_axis_name="c", subcore_axis_name="s",
                                    num_cores=NC, num_subcores=NS),
        scratch_types=[
            pltpu.VMEM((CHUNK * KNN,), jnp.int32),
            pltpu.VMEM((CHUNK * KNN, F), jnp.float32),
            pltpu.VMEM((2 * CHUNK, F), jnp.float32),
            pltpu.VMEM((1, F), jnp.float32),
            pltpu.SemaphoreType.DMA,
        ],
        compiler_params=pltpu.CompilerParams(use_tc_tiling_on_sc=False),
    )


def _sc_gather(tab, idx):
    return _sc_gather_fn()(tab, idx)


def _fin1_body(m_ref, ysum_ref, ysq_ref, g_ref, b_ref, wa_ref, wd_ref,
               x1_ref, a2_ref, c2_ref):
    mv = m_ref[...]
    cnt = jnp.float32(BN * KNN)
    mean = ysum_ref[...] / cnt
    var = ysq_ref[...] / cnt - mean * mean
    y = (mv - mean) / jnp.sqrt(var + EPS) * g_ref[...] + b_ref[...]
    x1 = _lrelu(y)
    x1_ref[...] = x1
    a2_ref[...] = jnp.dot(x1, wa_ref[...], preferred_element_type=jnp.float32)
    c2_ref[...] = jnp.dot(x1, wd_ref[...], preferred_element_type=jnp.float32)


def _fin1(m1, ysum, ysq, g1, b1, w2a, w2d):
    return pl.pallas_call(
        _fin1_body,
        out_shape=(jax.ShapeDtypeStruct((BN, F), jnp.float32),
                   jax.ShapeDtypeStruct((BN, F), jnp.float32),
                   jax.ShapeDtypeStruct((BN, F), jnp.float32)),
    )(m1, ysum, ysq, g1, b1, w2a, w2d)


def _stats2_body(s_ref, c_ref, q_ref, mean_ref, inv_ref):
    sv = s_ref[...]
    cv = c_ref[...]
    cnt = jnp.float32(BN * KNN)
    ssum = jnp.sum(sv, axis=0, keepdims=True)
    csum = jnp.sum(cv, axis=0, keepdims=True)
    cross = jnp.sum(cv * sv, axis=0, keepdims=True)
    c2s = jnp.sum(cv * cv, axis=0, keepdims=True)
    qsum = jnp.sum(q_ref[...], axis=0, keepdims=True)
    mean = (ssum + KNN * csum) / cnt
    ey2 = (qsum + 2.0 * cross + KNN * c2s) / cnt
    mean_ref[...] = mean
    inv_ref[...] = 1.0 / jnp.sqrt(ey2 - mean * mean + EPS)


def _stats2(s2, c2, q2):
    return pl.pallas_call(
        _stats2_body,
        out_shape=(jax.ShapeDtypeStruct((1, F), jnp.float32),
                   jax.ShapeDtypeStruct((1, F), jnp.float32)),
    )(s2, c2, q2)


def _fin2_body(x1_ref, m2_ref, c2_ref, mean_ref, inv_ref, g_ref, b_ref,
               wa_ref, wb_ref, ymax_ref, ysum_ref, ysq_ref):
    ti = pl.program_id(1)
    first = jnp.logical_and(pl.program_id(0) == 0, ti == 0)
    y2 = (m2_ref[0] + c2_ref[0] - mean_ref[...]) * inv_ref[...] * g_ref[...] \
        + b_ref[...]
    x2 = _lrelu(y2)
    y = jnp.dot(x1_ref[0], wa_ref[...], preferred_element_type=jnp.float32) \
        + jnp.dot(x2, wb_ref[...], preferred_element_type=jnp.float32)
    tmax = jnp.max(y, axis=0, keepdims=True)
    tsum = jnp.sum(y, axis=0, keepdims=True)
    tsq = jnp.sum(y * y, axis=0, keepdims=True)

    @pl.when(ti == 0)
    def _():
        ymax_ref[0] = tmax

    @pl.when(ti != 0)
    def _():
        ymax_ref[0] = jnp.maximum(ymax_ref[0], tmax)

    @pl.when(first)
    def _():
        ysum_ref[...] = tsum
        ysq_ref[...] = tsq

    @pl.when(jnp.logical_not(first))
    def _():
        ysum_ref[...] = ysum_ref[...] + tsum
        ysq_ref[...] = ysq_ref[...] + tsq


def _fin2(x1t3, m2t3, c2t3, mean2, inv2, g2, b2, w5a, w5b):
    return pl.pallas_call(
        _fin2_body,
        grid=(B, N // R_FIN),
        in_specs=[
            pl.BlockSpec((1, R_FIN, F), lambda b, t: (b, t, 0)),
            pl.BlockSpec((1, R_FIN, F), lambda b, t: (b, t, 0)),
            pl.BlockSpec((1, R_FIN, F), lambda b, t: (b, t, 0)),
            pl.BlockSpec((1, F), lambda b, t: (0, 0)),
            pl.BlockSpec((1, F), lambda b, t: (0, 0)),
            pl.BlockSpec((1, F), lambda b, t: (0, 0)),
            pl.BlockSpec((1, F), lambda b, t: (0, 0)),
            pl.BlockSpec((F, EMB), lambda b, t: (0, 0)),
            pl.BlockSpec((F, EMB), lambda b, t: (0, 0)),
        ],
        out_specs=(
            pl.BlockSpec((1, 1, EMB), lambda b, t: (b, 0, 0)),
            pl.BlockSpec((1, EMB), lambda b, t: (0, 0)),
            pl.BlockSpec((1, EMB), lambda b, t: (0, 0)),
        ),
        out_shape=(jax.ShapeDtypeStruct((B, 1, EMB), jnp.float32),
                   jax.ShapeDtypeStruct((1, EMB), jnp.float32),
                   jax.ShapeDtypeStruct((1, EMB), jnp.float32)),
    )(x1t3, m2t3, c2t3, mean2, inv2, g2, b2, w5a, w5b)


def _feat_body(ymax_ref, ysum_ref, ysq_ref, g_ref, b_ref, o_ref):
    cnt = jnp.float32(BN)
    mean = ysum_ref[...] / cnt
    var = ysq_ref[...] / cnt - mean * mean
    o_ref[...] = _lrelu((ymax_ref[...] - mean) / jnp.sqrt(var + EPS)
                        * g_ref[...] + b_ref[...])


def _feat(ymax, ysum, ysq, g5, b5):
    return pl.pallas_call(
        _feat_body,
        out_shape=jax.ShapeDtypeStruct((B, EMB), jnp.float32),
    )(ymax, ysum, ysq, g5, b5)


def kernel(x, W1, g1, b1, W2, g2, b2, W5, g5, b5):
    xt = jnp.transpose(x, (0, 2, 1))
    xtp = jnp.pad(xt, ((0, 0), (0, 0), (0, 13)))

    w1t8 = jnp.pad(W1.T, ((0, 2), (0, 0))).astype(jnp.bfloat16)
    w2a = W2[:, :F].T
    w2d = (W2[:, F:] - W2[:, :F]).T
    w5a = W5[:, :F].T
    w5b = W5[:, F:].T

    g1r, b1r = g1.reshape(1, F), b1.reshape(1, F)
    g2r, b2r = g2.reshape(1, F), b2.reshape(1, F)
    g5r, b5r = g5.reshape(1, EMB), b5.reshape(1, EMB)

    idx1 = _knn(xt, 3).reshape(BN * KNN)
    rows1 = _sc_gather_rows(xtp.reshape(BN, 16), idx1)
    m1, ysum1, ysq1 = _conv1(rows1.reshape(B, N * KNN, 16), xtp, w1t8)
    x1, a2, c2 = _fin1(m1.reshape(BN, F), ysum1, ysq1, g1r, b1r, w2a, w2d)

    x1t3 = x1.reshape(B, N, F)
    idx2 = _knn(x1t3, F).reshape(BN * KNN)
    m2, s2, q2 = _sc_gather(a2, idx2)
    mean2, inv2 = _stats2(s2, c2, q2)

    ymax, ysum, ysq = _fin2(x1t3, m2.reshape(B, N, F), c2.reshape(B, N, F),
                            mean2, inv2, g2r, b2r, w5a, w5b)
    return _feat(ymax.reshape(B, EMB), ysum, ysq, g5r, b5r)
